# Initial kernel scaffold; baseline (speedup 1.0000x reference)
import jax
import jax.numpy as jnp
from jax.experimental import pallas as pl


def kernel(x, W_up, b_up, W_imp, b_imp):
    raise NotImplementedError("write your pallas kernel here")


# speedup vs baseline: 35.7396x; 35.7396x over previous
import jax
import jax.numpy as jnp
from jax.experimental import pallas as pl
from jax.experimental.pallas import tpu as pltpu

D_MODEL = 1024
CAP = 64
BATCH = 4096
BM = 512
NB = BATCH // BM

NEG = float(jnp.finfo(jnp.float32).min)


def _proj_kernel(x_ref, wup_ref, bup_ref, wimp_ref, bimp_ref, u_ref, p_ref):
    x = x_ref[...]
    u = jax.lax.dot_general(x, wup_ref[...], (((1,), (1,)), ((), ())),
                            preferred_element_type=jnp.float32)
    u_ref[...] = u + bup_ref[...]
    z = jax.lax.dot_general(wimp_ref[...], x, (((1,), (1,)), ((), ())),
                            preferred_element_type=jnp.float32)
    p = jax.nn.sigmoid(z + bimp_ref[0, 0])
    p_ref[...] = p.reshape(1, 1, BM)


def _mem_kernel(p_ref, u_ref, out_ref, slots_ref, imp_ref, ages_ref):
    b = pl.program_id(0)

    @pl.when(b == 0)
    def _init():
        slots_ref[...] = jnp.zeros((CAP, D_MODEL), jnp.float32)
        imp_ref[...] = jnp.zeros((1, CAP), jnp.float32)
        ages_ref[...] = jnp.zeros((1, CAP), jnp.float32)

    lane = jax.lax.broadcasted_iota(jnp.int32, (1, CAP), 1)

    def body(i, carry):
        t = b * BM + i
        p = p_ref[b, 0, i]
        imp = imp_ref[...]
        ages = ages_ref[...]
        m = jnp.min(imp)
        amin = jnp.min(jnp.where(imp == m, lane, CAP))
        fill = t < CAP
        idx = jnp.where(fill, t, amin)
        should = jnp.logical_or(fill, p > 0.1)
        u_row = u_ref[pl.ds(i, 1), :]
        old = slots_ref[pl.ds(idx, 1), :]
        slots_ref[pl.ds(idx, 1), :] = jnp.where(should, u_row, old)
        sel = jnp.logical_and(lane == idx, should)
        imp = jnp.where(sel, jnp.maximum(0.1, p), imp)
        ages = jnp.where(sel, 0.0, ages) + 1.0
        imp = jnp.where(ages > 50.0, imp * 0.9, imp)
        imp_ref[...] = imp
        ages_ref[...] = ages
        occ = lane <= t
        logits = jnp.where(occ, imp, NEG)
        mx = jnp.max(logits)
        e = jnp.exp(logits - mx)
        w = e / jnp.sum(e)
        row = jax.lax.dot_general(w, slots_ref[...], (((1,), (0,)), ((), ())),
                                  preferred_element_type=jnp.float32)
        out_ref[pl.ds(i, 1), :] = row
        return carry

    jax.lax.fori_loop(0, BM, body, 0)


def _stage_a(x, W_up, b_up, W_imp, b_imp, interpret=False):
    return pl.pallas_call(
        _proj_kernel,
        grid=(NB,),
        in_specs=[
            pl.BlockSpec((BM, D_MODEL), lambda b: (b, 0)),
            pl.BlockSpec((D_MODEL, D_MODEL), lambda b: (0, 0)),
            pl.BlockSpec((1, D_MODEL), lambda b: (0, 0)),
            pl.BlockSpec((1, D_MODEL), lambda b: (0, 0)),
            pl.BlockSpec(memory_space=pltpu.SMEM),
        ],
        out_specs=[
            pl.BlockSpec((BM, D_MODEL), lambda b: (b, 0)),
            pl.BlockSpec((1, 1, BM), lambda b: (b, 0, 0)),
        ],
        out_shape=[
            jax.ShapeDtypeStruct((BATCH, D_MODEL), jnp.float32),
            jax.ShapeDtypeStruct((NB, 1, BM), jnp.float32),
        ],
        interpret=interpret,
    )(x, W_up, b_up.reshape(1, D_MODEL), W_imp, b_imp.reshape(1, 1))


def _stage_b(P, U, interpret=False):
    return pl.pallas_call(
        _mem_kernel,
        grid=(NB,),
        in_specs=[
            pl.BlockSpec(memory_space=pltpu.SMEM),
            pl.BlockSpec((BM, D_MODEL), lambda b: (b, 0)),
        ],
        out_specs=pl.BlockSpec((BM, D_MODEL), lambda b: (b, 0)),
        out_shape=jax.ShapeDtypeStruct((BATCH, D_MODEL), jnp.float32),
        scratch_shapes=[
            pltpu.VMEM((CAP, D_MODEL), jnp.float32),
            pltpu.VMEM((1, CAP), jnp.float32),
            pltpu.VMEM((1, CAP), jnp.float32),
        ],
        interpret=interpret,
    )(P, U)


def kernel(x, W_up, b_up, W_imp, b_imp):
    U, P = _stage_a(x, W_up, b_up, W_imp, b_imp)
    return _stage_b(P, U)


# speedup vs baseline: 94.0675x; 2.6320x over previous
import jax
import jax.numpy as jnp
from jax.experimental import pallas as pl
from jax.experimental.pallas import tpu as pltpu

D_MODEL = 1024
CAP = 64
BATCH = 4096
BMA = 512
NBA = BATCH // BMA
BM = 256
NB = BATCH // BM

NEG = float(jnp.finfo(jnp.float32).min)


def _proj_kernel(x_ref, wup_ref, bup_ref, wimp_ref, bimp_ref, u_ref, p_ref):
    x = x_ref[...]
    u = jax.lax.dot_general(x, wup_ref[...], (((1,), (1,)), ((), ())),
                            preferred_element_type=jnp.float32)
    u_ref[...] = u + bup_ref[...]
    z = jax.lax.dot_general(wimp_ref[...], x, (((1,), (1,)), ((), ())),
                            preferred_element_type=jnp.float32)
    p = jax.nn.sigmoid(z + bimp_ref[0, 0])
    p_ref[...] = p.reshape(1, 1, BMA)


def _mem_kernel(p_ref, u_ref, out_ref, slots_ref, state_ref, imp_rows, lw_rows):
    b = pl.program_id(0)
    t0 = b * BM

    @pl.when(b == 0)
    def _init():
        slots_ref[...] = jnp.zeros((CAP, D_MODEL), jnp.float32)
        state_ref[0:1, :] = jnp.zeros((1, CAP), jnp.float32)
        state_ref[1:2, :] = jnp.zeros((1, CAP), jnp.float32)
        state_ref[2:3, :] = jnp.full((1, CAP), -1.0, jnp.float32)

    lane = jax.lax.broadcasted_iota(jnp.int32, (1, CAP), 1)
    lane_f = lane.astype(jnp.float32)

    def body(i, carry):
        imp, ages, lw = carry
        t = t0 + i
        t_f = t.astype(jnp.float32)
        p = p_ref[b, 0, i]
        m = jnp.min(imp, axis=1, keepdims=True)
        cand = jnp.where(imp == m, lane, CAP)
        amin = jnp.min(cand, axis=1, keepdims=True)
        fill = t < CAP
        idx = jnp.where(fill, t, amin)
        should = jnp.logical_or(fill, p > 0.1)
        sel = jnp.logical_and(lane == idx, should)
        imp = jnp.where(sel, jnp.maximum(0.1, p), imp)
        ages = jnp.where(sel, 0.0, ages) + 1.0
        imp = jnp.where(ages > 50.0, imp * 0.9, imp)
        lw = jnp.where(sel, t_f, lw)
        imp_rows[pl.ds(i, 1), :] = imp
        lw_rows[pl.ds(i, 1), :] = lw
        return imp, ages, lw

    imp0 = state_ref[0:1, :]
    ages0 = state_ref[1:2, :]
    lw0 = state_ref[2:3, :]
    impf, agesf, lwf = jax.lax.fori_loop(0, BM, body, (imp0, ages0, lw0))
    state_ref[0:1, :] = impf
    state_ref[1:2, :] = agesf
    state_ref[2:3, :] = lwf

    IMP = imp_rows[...]
    LW = lw_rows[...]
    occ = LW >= 0.0
    logits = jnp.where(occ, IMP, NEG)
    mx = jnp.max(logits, axis=1, keepdims=True)
    e = jnp.exp(logits - mx)
    w = e / jnp.sum(e, axis=1, keepdims=True)

    LWi = LW.astype(jnp.int32)
    in_blk = LWi >= t0
    M1 = jnp.where(in_blk, 0.0, w)
    krow = jax.lax.broadcasted_iota(jnp.int32, (BM, BM), 1) + t0
    M2 = jnp.zeros((BM, BM), jnp.float32)
    for j in range(CAP):
        M2 = M2 + jnp.where(LWi[:, j:j + 1] == krow, w[:, j:j + 1], 0.0)
    U = u_ref[...]
    out = jax.lax.dot_general(M1, slots_ref[...], (((1,), (0,)), ((), ())),
                              preferred_element_type=jnp.float32)
    out = out + jax.lax.dot_general(M2, U, (((1,), (0,)), ((), ())),
                                    preferred_element_type=jnp.float32)
    out_ref[...] = out

    lwi = lwf.astype(jnp.int32)
    ST = (lwi == (jax.lax.broadcasted_iota(jnp.int32, (BM, CAP), 0)
                  + t0)).astype(jnp.float32)
    G = jax.lax.dot_general(ST, U, (((0,), (0,)), ((), ())),
                            preferred_element_type=jnp.float32)
    ir = jax.lax.broadcasted_iota(jnp.int32, (CAP, CAP), 0)
    ic = jax.lax.broadcasted_iota(jnp.int32, (CAP, CAP), 1)
    keep = jnp.logical_and(ir == ic, lwi < t0).astype(jnp.float32)
    slots_ref[...] = G + jax.lax.dot_general(
        keep, slots_ref[...], (((1,), (0,)), ((), ())),
        preferred_element_type=jnp.float32)


def _stage_a(x, W_up, b_up, W_imp, b_imp, interpret=False):
    return pl.pallas_call(
        _proj_kernel,
        grid=(NBA,),
        in_specs=[
            pl.BlockSpec((BMA, D_MODEL), lambda b: (b, 0)),
            pl.BlockSpec((D_MODEL, D_MODEL), lambda b: (0, 0)),
            pl.BlockSpec((1, D_MODEL), lambda b: (0, 0)),
            pl.BlockSpec((1, D_MODEL), lambda b: (0, 0)),
            pl.BlockSpec(memory_space=pltpu.SMEM),
        ],
        out_specs=[
            pl.BlockSpec((BMA, D_MODEL), lambda b: (b, 0)),
            pl.BlockSpec((1, 1, BMA), lambda b: (b, 0, 0)),
        ],
        out_shape=[
            jax.ShapeDtypeStruct((BATCH, D_MODEL), jnp.float32),
            jax.ShapeDtypeStruct((NBA, 1, BMA), jnp.float32),
        ],
        interpret=interpret,
    )(x, W_up, b_up.reshape(1, D_MODEL), W_imp, b_imp.reshape(1, 1))


def _stage_b(P, U, interpret=False):
    return pl.pallas_call(
        _mem_kernel,
        grid=(NB,),
        in_specs=[
            pl.BlockSpec(memory_space=pltpu.SMEM),
            pl.BlockSpec((BM, D_MODEL), lambda b: (b, 0)),
        ],
        out_specs=pl.BlockSpec((BM, D_MODEL), lambda b: (b, 0)),
        out_shape=jax.ShapeDtypeStruct((BATCH, D_MODEL), jnp.float32),
        scratch_shapes=[
            pltpu.VMEM((CAP, D_MODEL), jnp.float32),
            pltpu.VMEM((8, CAP), jnp.float32),
            pltpu.VMEM((BM, CAP), jnp.float32),
            pltpu.VMEM((BM, CAP), jnp.float32),
        ],
        interpret=interpret,
    )(P, U)


def kernel(x, W_up, b_up, W_imp, b_imp):
    U, P = _stage_a(x, W_up, b_up, W_imp, b_imp)
    return _stage_b(P.reshape(NB, 1, BM), U)
